# Initial kernel scaffold; baseline (speedup 1.0000x reference)
import jax
import jax.numpy as jnp
from jax.experimental import pallas as pl


def kernel(h_V, h_E, E_idx, W1, b1, W2, b2, W3, b3, W11, b11, W12, b12, W13, b13, W_in, b_in, W_out, b_out, g1, bt1, g2, bt2, g3, bt3):
    raise NotImplementedError("write your pallas kernel here")


# speedup vs baseline: 4.5370x; 4.5370x over previous
import functools

import jax
import jax.numpy as jnp
from jax import lax
from jax.experimental import pallas as pl
from jax.experimental.pallas import tpu as pltpu
from jax.experimental.pallas import tpu_sc as plsc

H = 128
SCALE = 30.0

_NC = 2
_NS = 16
_NW = _NC * _NS

_CHUNK = 128


def _gelu(x):
    return 0.5 * x * (1.0 + lax.erf(x * 0.7071067811865476))


def _ln(x, g, b, eps=1e-5):
    mu = jnp.mean(x, axis=-1, keepdims=True)
    var = jnp.mean((x - mu) ** 2, axis=-1, keepdims=True)
    return (x - mu) / jnp.sqrt(var + eps) * g + b


def _proj_body(hv_ref, w_ref, o_ref):
    o_ref[:] = jnp.dot(hv_ref[:], w_ref[:], preferred_element_type=jnp.float32)


def _project(h_V, W):
    L = h_V.shape[0]
    return pl.pallas_call(
        _proj_body,
        out_shape=jax.ShapeDtypeStruct((L, H), jnp.float32),
    )(h_V, W)


def _sc_gather(table, idx_flat):
    B = idx_flat.shape[0]
    assert B % _NW == 0
    bpw = B // _NW
    n_full = bpw // _CHUNK
    rem = bpw - n_full * _CHUNK
    assert bpw % 8 == 0 and rem % 8 == 0

    mesh = plsc.VectorSubcoreMesh(core_axis_name="c", subcore_axis_name="s")

    @functools.partial(
        pl.kernel,
        mesh=mesh,
        out_type=jax.ShapeDtypeStruct((B, H), jnp.float32),
        scratch_types=[
            pltpu.VMEM((_CHUNK,), jnp.int32),
            pltpu.VMEM((_CHUNK, H), jnp.float32),
            pltpu.SemaphoreType.DMA,
        ],
    )
    def k(table_hbm, idx_hbm, out_hbm, idx_v, rows_v, sem):
        wid = lax.axis_index("s") * _NC + lax.axis_index("c")
        base = wid * bpw

        def body(g, carry):
            off = base + g * _CHUNK
            pltpu.sync_copy(idx_hbm.at[pl.ds(off, _CHUNK)], idx_v)
            pltpu.async_copy(table_hbm.at[idx_v], rows_v, sem).wait()
            pltpu.sync_copy(rows_v, out_hbm.at[pl.ds(off, _CHUNK)])
            return carry

        lax.fori_loop(0, n_full, body, 0)
        if rem:
            off = base + n_full * _CHUNK
            idx_r = idx_v.at[pl.ds(0, rem)]
            rows_r = rows_v.at[pl.ds(0, rem)]
            pltpu.sync_copy(idx_hbm.at[pl.ds(off, rem)], idx_r)
            pltpu.async_copy(table_hbm.at[idx_r], rows_r, sem).wait()
            pltpu.sync_copy(rows_r, out_hbm.at[pl.ds(off, rem)])

    return k(table, idx_flat)


def _tc1_body(TL, K,
              hv_ref, he_ref, g1_ref,
              w1a, w1b, b1, w2, b2, w3, b3,
              w_in, b_in, w_out, b_out,
              g1g, bt1, g2g, bt2,
              w11a, w11c, b11,
              o_v, o_a2, o_c2):
    v = hv_ref[:]
    e = he_ref[:]
    g = g1_ref[:]
    a1 = jnp.dot(v, w1a[:], preferred_element_type=jnp.float32) + b1[:]
    pre = jnp.dot(e, w1b[:], preferred_element_type=jnp.float32) + g
    pre = (pre.reshape(TL, K, H) + a1[:, None, :]).reshape(TL * K, H)
    m = _gelu(pre)
    m = _gelu(jnp.dot(m, w2[:], preferred_element_type=jnp.float32) + b2[:])
    m = jnp.dot(m, w3[:], preferred_element_type=jnp.float32) + b3[:]
    dh = jnp.sum(m.reshape(TL, K, H), axis=1) * (1.0 / SCALE)
    x = _ln(v + dh, g1g[:], bt1[:])
    t = jnp.dot(_gelu(jnp.dot(x, w_in[:], preferred_element_type=jnp.float32) + b_in[:]),
                w_out[:], preferred_element_type=jnp.float32) + b_out[:]
    x = _ln(x + t, g2g[:], bt2[:])
    o_v[:] = x
    o_a2[:] = jnp.dot(x, w11a[:], preferred_element_type=jnp.float32) + b11[:]
    o_c2[:] = jnp.dot(x, w11c[:], preferred_element_type=jnp.float32)


def _tc2_body(TL, K,
              he_ref, g2_ref, a2_ref,
              w11b, w12, b12, w13, b13,
              g3g, bt3,
              o_e):
    e = he_ref[:]
    g = g2_ref[:]
    a2 = a2_ref[:]
    pre = jnp.dot(e, w11b[:], preferred_element_type=jnp.float32) + g
    pre = (pre.reshape(TL, K, H) + a2[:, None, :]).reshape(TL * K, H)
    m = _gelu(pre)
    m = _gelu(jnp.dot(m, w12[:], preferred_element_type=jnp.float32) + b12[:])
    m = jnp.dot(m, w13[:], preferred_element_type=jnp.float32) + b13[:]
    o_e[:] = _ln(e + m, g3g[:], bt3[:])


def _row(b):
    return b.reshape(1, -1)


def kernel(h_V, h_E, E_idx, W1, b1, W2, b2, W3, b3, W11, b11, W12, b12, W13,
           b13, W_in, b_in, W_out, b_out, g1, bt1, g2, bt2, g3, bt3):
    L, K = h_E.shape[0], h_E.shape[1]
    TL = 200
    assert L % TL == 0
    grid = L // TL

    W1a, W1b, W1c = W1[:H], W1[H:2 * H], W1[2 * H:]
    W11a, W11b, W11c = W11[:H], W11[H:2 * H], W11[2 * H:]

    he2 = h_E.reshape(L * K, H)
    idx_flat = E_idx.reshape(L * K)

    c1 = _project(h_V, W1c)
    gth1 = _sc_gather(c1, idx_flat)

    node_spec = pl.BlockSpec((TL, H), lambda i: (i, 0))
    edge_spec = pl.BlockSpec((TL * K, H), lambda i: (i, 0))

    def wspec(a):
        return pl.BlockSpec(a.shape, lambda i: tuple(0 for _ in a.shape))

    weights1 = (W1a, W1b, _row(b1), W2, _row(b2), W3, _row(b3),
                W_in, _row(b_in), W_out, _row(b_out),
                _row(g1), _row(bt1), _row(g2), _row(bt2),
                W11a, W11c, _row(b11))
    h_V2, a2, c2 = pl.pallas_call(
        functools.partial(_tc1_body, TL, K),
        grid=(grid,),
        in_specs=[node_spec, edge_spec, edge_spec] + [wspec(w) for w in weights1],
        out_specs=[node_spec, node_spec, node_spec],
        out_shape=[jax.ShapeDtypeStruct((L, H), jnp.float32)] * 3,
    )(h_V, he2, gth1, *weights1)

    gth2 = _sc_gather(c2, idx_flat)

    weights2 = (W11b, W12, _row(b12), W13, _row(b13), _row(g3), _row(bt3))
    h_E2 = pl.pallas_call(
        functools.partial(_tc2_body, TL, K),
        grid=(grid,),
        in_specs=[edge_spec, edge_spec, node_spec] + [wspec(w) for w in weights2],
        out_specs=edge_spec,
        out_shape=jax.ShapeDtypeStruct((L * K, H), jnp.float32),
    )(he2, gth2, a2, *weights2)

    return h_V2, h_E2.reshape(L, K, H)
